# baseline (speedup 1.0000x reference)
import jax
import jax.numpy as jnp
from jax import lax
from jax.experimental import pallas as pl
from jax.experimental.pallas import tpu as pltpu
from jax.experimental.pallas import tpu_sc as plsc

ROWS = 16384
SEQ = 200
EMB = 64
VOCAB = 8

NPAIR = ROWS * SEQ // 2
NC = 2
NS = 16
NW = NC * NS
PER_W = NPAIR // NW
CHUNK = 256
NCHUNKS = PER_W // CHUNK


def _sc_body(idx_hbm, w2_hbm, out_hbm,
             w2_v, idx0, idx1, rows0, rows1,
             sem_w, sem_i0, sem_i1, sem_g0, sem_g1, sem_o0, sem_o1):
    wid = lax.axis_index("s") * NC + lax.axis_index("c")
    base = wid * PER_W
    idx = (idx0, idx1)
    rows = (rows0, rows1)
    sem_i = (sem_i0, sem_i1)
    sem_g = (sem_g0, sem_g1)
    sem_o = (sem_o0, sem_o1)

    @pl.when(lax.axis_index("s") == 0)
    def _():
        pltpu.make_async_copy(w2_hbm, w2_v, sem_w).start()

    for b in range(2):
        pltpu.make_async_copy(
            idx_hbm.at[pl.ds(base + b * CHUNK, CHUNK)], idx[b], sem_i[b]
        ).start()

    @pl.when(lax.axis_index("s") == 0)
    def _():
        pltpu.make_async_copy(w2_hbm, w2_v, sem_w).wait()

    plsc.subcore_barrier()

    def step(it, _):
        j0 = it * 2
        for b in range(2):
            j = j0 + b
            off = base + j * CHUNK

            @pl.when(j >= 2)
            def _():
                pltpu.make_async_copy(
                    rows[b], out_hbm.at[pl.ds(off - 2 * CHUNK, CHUNK)], sem_o[b]
                ).wait()

            pltpu.make_async_copy(
                idx_hbm.at[pl.ds(off, CHUNK)], idx[b], sem_i[b]
            ).wait()

            g = pltpu.make_async_copy(w2_v.at[idx[b]], rows[b], sem_g[b])
            g.start()
            g.wait()

            pltpu.make_async_copy(
                rows[b], out_hbm.at[pl.ds(off, CHUNK)], sem_o[b]
            ).start()

            @pl.when(j + 2 < NCHUNKS)
            def _():
                pltpu.make_async_copy(
                    idx_hbm.at[pl.ds(off + 2 * CHUNK, CHUNK)], idx[b], sem_i[b]
                ).start()
        return _

    lax.fori_loop(0, NCHUNKS // 2, step, None)

    for b in range(2):
        off = base + (NCHUNKS - 2 + b) * CHUNK
        pltpu.make_async_copy(
            rows[b], out_hbm.at[pl.ds(off, CHUNK)], sem_o[b]
        ).wait()


@jax.jit
def _sc_lookup(idx2, W2):
    mesh = plsc.VectorSubcoreMesh(core_axis_name="c", subcore_axis_name="s")
    return pl.kernel(
        _sc_body,
        out_type=jax.ShapeDtypeStruct((NPAIR, 2 * EMB), jnp.float32),
        mesh=mesh,
        scratch_types=[
            pltpu.VMEM_SHARED((VOCAB * VOCAB, 2 * EMB), jnp.float32),
            pltpu.VMEM((CHUNK,), jnp.int32),
            pltpu.VMEM((CHUNK,), jnp.int32),
            pltpu.VMEM((CHUNK, 2 * EMB), jnp.float32),
            pltpu.VMEM((CHUNK, 2 * EMB), jnp.float32),
            pltpu.SemaphoreType.DMA,
            pltpu.SemaphoreType.DMA,
            pltpu.SemaphoreType.DMA,
            pltpu.SemaphoreType.DMA,
            pltpu.SemaphoreType.DMA,
            pltpu.SemaphoreType.DMA,
            pltpu.SemaphoreType.DMA,
        ],
    )(idx2, W2)


def kernel(x, W):
    xp = x.reshape(ROWS, SEQ // 2, 2)
    idx2 = (xp[:, :, 0] * VOCAB + xp[:, :, 1]).reshape(NPAIR)
    W2 = jnp.concatenate(
        [
            jnp.broadcast_to(W[:, None, :], (VOCAB, VOCAB, EMB)),
            jnp.broadcast_to(W[None, :, :], (VOCAB, VOCAB, EMB)),
        ],
        axis=-1,
    ).reshape(VOCAB * VOCAB, 2 * EMB)
    out = _sc_lookup(idx2, W2)
    return out.reshape(ROWS, SEQ, EMB)


# speedup vs baseline: 1.2346x; 1.2346x over previous
import jax
import jax.numpy as jnp
from jax import lax
from jax.experimental import pallas as pl
from jax.experimental.pallas import tpu as pltpu
from jax.experimental.pallas import tpu_sc as plsc

ROWS = 16384
SEQ = 200
EMB = 64
VOCAB = 8

NC = 2
NS = 16
NW = NC * NS
IPW = ROWS // NW
NVB = IPW // 16


def _sc_body(xt_hbm, wt_hbm, out_hbm,
             wt_v, idx0, idx1, ob0, ob1,
             sem_w, sem_i0, sem_i1, sem_o0, sem_o1):
    wid = lax.axis_index("s") * NC + lax.axis_index("c")
    i0 = wid * IPW
    idx = (idx0, idx1)
    ob = (ob0, ob1)
    sem_i = (sem_i0, sem_i1)
    sem_o = (sem_o0, sem_o1)

    pltpu.make_async_copy(wt_hbm, wt_v, sem_w).start()
    for b in range(2):
        pltpu.make_async_copy(
            xt_hbm.at[pl.ds(b * ROWS + i0, IPW)], idx[b], sem_i[b]
        ).start()
    pltpu.make_async_copy(wt_hbm, wt_v, sem_w).wait()

    def step(it, _):
        j0 = it * 2
        for b in range(2):
            j = j0 + b

            pltpu.make_async_copy(
                xt_hbm.at[pl.ds(j * ROWS + i0, IPW)], idx[b], sem_i[b]
            ).wait()

            @pl.when(j >= 2)
            def _():
                pltpu.make_async_copy(
                    ob[b],
                    out_hbm.at[pl.ds((j - 2) * EMB, EMB), pl.ds(i0, IPW)],
                    sem_o[b],
                ).wait()

            def expand(vb, _):
                o = pl.multiple_of(vb * 16, 16)
                g = idx[b][pl.ds(o, 16)]
                for k in range(EMB):
                    wk = wt_v[pl.ds(k * 16, 16)]
                    ob[b][k, pl.ds(o, 16)] = jnp.take_along_axis(wk, g, axis=0)
                return _

            lax.fori_loop(0, NVB, expand, None, unroll=False)

            pltpu.make_async_copy(
                ob[b],
                out_hbm.at[pl.ds(j * EMB, EMB), pl.ds(i0, IPW)],
                sem_o[b],
            ).start()

            @pl.when(j + 2 < SEQ)
            def _():
                pltpu.make_async_copy(
                    xt_hbm.at[pl.ds((j + 2) * ROWS + i0, IPW)], idx[b], sem_i[b]
                ).start()
        return _

    lax.fori_loop(0, SEQ // 2, step, None)

    for b in range(2):
        j = SEQ - 2 + b
        pltpu.make_async_copy(
            ob[b],
            out_hbm.at[pl.ds(j * EMB, EMB), pl.ds(i0, IPW)],
            sem_o[b],
        ).wait()


@jax.jit
def _sc_lookup(xt_flat, WT):
    mesh = plsc.VectorSubcoreMesh(core_axis_name="c", subcore_axis_name="s")
    return pl.kernel(
        _sc_body,
        out_type=jax.ShapeDtypeStruct((SEQ * EMB, ROWS), jnp.float32),
        mesh=mesh,
        scratch_types=[
            pltpu.VMEM((EMB * 16,), jnp.float32),
            pltpu.VMEM((IPW,), jnp.int32),
            pltpu.VMEM((IPW,), jnp.int32),
            pltpu.VMEM((EMB, IPW), jnp.float32),
            pltpu.VMEM((EMB, IPW), jnp.float32),
            pltpu.SemaphoreType.DMA,
            pltpu.SemaphoreType.DMA,
            pltpu.SemaphoreType.DMA,
            pltpu.SemaphoreType.DMA,
            pltpu.SemaphoreType.DMA,
        ],
    )(xt_flat, WT)


def kernel(x, W):
    xt_flat = x.T.reshape(SEQ * ROWS)
    WT = jnp.zeros((EMB, 16), jnp.float32).at[:, :VOCAB].set(W.T).reshape(EMB * 16)
    o2 = _sc_lookup(xt_flat, WT)
    return o2.reshape(SEQ, EMB, ROWS).transpose(2, 0, 1)


# speedup vs baseline: 6.2666x; 5.0760x over previous
import jax
import jax.numpy as jnp
from jax import lax
from jax.experimental import pallas as pl
from jax.experimental.pallas import tpu as pltpu
from jax.experimental.pallas import tpu_sc as plsc

ROWS = 16384
SEQ = 200
EMB = 64
VOCAB = 8

NC = 2
NS = 16
NW = NC * NS
IPW = ROWS // NW
NVB = IPW // 16


def _sc_body(xt_hbm, wt_hbm, out_hbm,
             wt_v, idx0, idx1, ob0, ob1,
             sem_w, sem_i0, sem_i1, sem_o0, sem_o1):
    wid = lax.axis_index("s") * NC + lax.axis_index("c")
    i0 = wid * IPW
    idx = (idx0, idx1)
    ob = (ob0, ob1)
    sem_i = (sem_i0, sem_i1)
    sem_o = (sem_o0, sem_o1)

    pltpu.make_async_copy(wt_hbm, wt_v, sem_w).start()
    for b in range(2):
        pltpu.make_async_copy(
            xt_hbm.at[pl.ds(b * ROWS + i0, IPW)], idx[b], sem_i[b]
        ).start()
    pltpu.make_async_copy(wt_hbm, wt_v, sem_w).wait()

    def step(it, _):
        j0 = it * 2
        for b in range(2):
            j = j0 + b

            pltpu.make_async_copy(
                xt_hbm.at[pl.ds(j * ROWS + i0, IPW)], idx[b], sem_i[b]
            ).wait()

            @pl.when(j >= 2)
            def _():
                pltpu.make_async_copy(
                    ob[b],
                    out_hbm.at[pl.ds((j - 2) * EMB, EMB), pl.ds(i0, IPW)],
                    sem_o[b],
                ).wait()

            for h in range(2):
                gs = [
                    idx[b][pl.ds(h * IPW // 2 + c * 16, 16)]
                    for c in range(NVB // 2)
                ]

                def expand(k, _):
                    ko = pl.multiple_of(k * 16, 16)
                    wk = wt_v[pl.ds(ko, 16)]
                    for c, g in enumerate(gs):
                        ob[b][k, pl.ds(h * IPW // 2 + c * 16, 16)] = (
                            jnp.take_along_axis(wk, g, axis=0)
                        )
                    return _

                lax.fori_loop(0, EMB, expand, None, unroll=False)

            pltpu.make_async_copy(
                ob[b],
                out_hbm.at[pl.ds(j * EMB, EMB), pl.ds(i0, IPW)],
                sem_o[b],
            ).start()

            @pl.when(j + 2 < SEQ)
            def _():
                pltpu.make_async_copy(
                    xt_hbm.at[pl.ds((j + 2) * ROWS + i0, IPW)], idx[b], sem_i[b]
                ).start()
        return _

    lax.fori_loop(0, SEQ // 2, step, None)

    for b in range(2):
        j = SEQ - 2 + b
        pltpu.make_async_copy(
            ob[b],
            out_hbm.at[pl.ds(j * EMB, EMB), pl.ds(i0, IPW)],
            sem_o[b],
        ).wait()


@jax.jit
def _sc_lookup(xt_flat, WT):
    mesh = plsc.VectorSubcoreMesh(core_axis_name="c", subcore_axis_name="s")
    return pl.kernel(
        _sc_body,
        out_type=jax.ShapeDtypeStruct((SEQ * EMB, ROWS), jnp.float32),
        mesh=mesh,
        scratch_types=[
            pltpu.VMEM((EMB * 16,), jnp.float32),
            pltpu.VMEM((IPW,), jnp.int32),
            pltpu.VMEM((IPW,), jnp.int32),
            pltpu.VMEM((EMB, IPW), jnp.float32),
            pltpu.VMEM((EMB, IPW), jnp.float32),
            pltpu.SemaphoreType.DMA,
            pltpu.SemaphoreType.DMA,
            pltpu.SemaphoreType.DMA,
            pltpu.SemaphoreType.DMA,
            pltpu.SemaphoreType.DMA,
        ],
    )(xt_flat, WT)


def kernel(x, W):
    xt_flat = x.T.reshape(SEQ * ROWS)
    WT = jnp.zeros((EMB, 16), jnp.float32).at[:, :VOCAB].set(W.T).reshape(EMB * 16)
    o2 = _sc_lookup(xt_flat, WT)
    return o2.reshape(SEQ, EMB, ROWS).transpose(2, 0, 1)


# speedup vs baseline: 6.2689x; 1.0004x over previous
import jax
import jax.numpy as jnp
from jax import lax
from jax.experimental import pallas as pl
from jax.experimental.pallas import tpu as pltpu
from jax.experimental.pallas import tpu_sc as plsc

ROWS = 16384
SEQ = 200
EMB = 64
VOCAB = 8

NC = 2
NS = 16
NW = NC * NS
IPW = ROWS // NW
NVB = IPW // 16


def _sc_body(xt_hbm, wt_hbm, out_hbm,
             wt_v, idx0, idx1, ob0, ob1,
             sem_w, sem_i0, sem_i1, sem_o0, sem_o1):
    wid = lax.axis_index("s") * NC + lax.axis_index("c")
    i0 = wid * IPW
    idx = (idx0, idx1)
    ob = (ob0, ob1)
    sem_i = (sem_i0, sem_i1)
    sem_o = (sem_o0, sem_o1)

    pltpu.make_async_copy(wt_hbm, wt_v, sem_w).start()
    for b in range(2):
        pltpu.make_async_copy(
            xt_hbm.at[pl.ds(b * ROWS + i0, IPW)], idx[b], sem_i[b]
        ).start()
    pltpu.make_async_copy(wt_hbm, wt_v, sem_w).wait()

    def step(it, _):
        j0 = it * 2
        for b in range(2):
            j = j0 + b

            pltpu.make_async_copy(
                xt_hbm.at[pl.ds(j * ROWS + i0, IPW)], idx[b], sem_i[b]
            ).wait()

            @pl.when(j >= 2)
            def _():
                pltpu.make_async_copy(
                    ob[b],
                    out_hbm.at[pl.ds((j - 2) * EMB, EMB), pl.ds(i0, IPW)],
                    sem_o[b],
                ).wait()

            for h in range(2):
                gs = [
                    idx[b][pl.ds(h * IPW // 2 + c * 16, 16)]
                    for c in range(NVB // 2)
                ]

                def expand(k, _):
                    ko = pl.multiple_of(k * 16, 16)
                    wk = wt_v[pl.ds(ko, 16)]
                    for c, g in enumerate(gs):
                        ob[b][k, pl.ds(h * IPW // 2 + c * 16, 16)] = (
                            jnp.take_along_axis(wk, g, axis=0)
                        )
                    return _

                lax.fori_loop(0, EMB, expand, None, unroll=2)

            pltpu.make_async_copy(
                ob[b],
                out_hbm.at[pl.ds(j * EMB, EMB), pl.ds(i0, IPW)],
                sem_o[b],
            ).start()

            @pl.when(j + 2 < SEQ)
            def _():
                pltpu.make_async_copy(
                    xt_hbm.at[pl.ds((j + 2) * ROWS + i0, IPW)], idx[b], sem_i[b]
                ).start()
        return _

    lax.fori_loop(0, SEQ // 2, step, None)

    for b in range(2):
        j = SEQ - 2 + b
        pltpu.make_async_copy(
            ob[b],
            out_hbm.at[pl.ds(j * EMB, EMB), pl.ds(i0, IPW)],
            sem_o[b],
        ).wait()


@jax.jit
def _sc_lookup(xt_flat, WT):
    mesh = plsc.VectorSubcoreMesh(core_axis_name="c", subcore_axis_name="s")
    return pl.kernel(
        _sc_body,
        out_type=jax.ShapeDtypeStruct((SEQ * EMB, ROWS), jnp.float32),
        mesh=mesh,
        scratch_types=[
            pltpu.VMEM((EMB * 16,), jnp.float32),
            pltpu.VMEM((IPW,), jnp.int32),
            pltpu.VMEM((IPW,), jnp.int32),
            pltpu.VMEM((EMB, IPW), jnp.float32),
            pltpu.VMEM((EMB, IPW), jnp.float32),
            pltpu.SemaphoreType.DMA,
            pltpu.SemaphoreType.DMA,
            pltpu.SemaphoreType.DMA,
            pltpu.SemaphoreType.DMA,
            pltpu.SemaphoreType.DMA,
        ],
    )(xt_flat, WT)


def kernel(x, W):
    xt_flat = x.T.reshape(SEQ * ROWS)
    WT = jnp.zeros((EMB, 16), jnp.float32).at[:, :VOCAB].set(W.T).reshape(EMB * 16)
    o2 = _sc_lookup(xt_flat, WT)
    return o2.reshape(SEQ, EMB, ROWS).transpose(2, 0, 1)
